# baseline (speedup 1.0000x reference)
import dataclasses
import functools

import jax
import jax.numpy as jnp
import numpy as np
from jax import lax
from jax.experimental import pallas as pl
from jax.experimental.pallas import tpu as pltpu
from jax.experimental.pallas import tpu_sc as plsc

N_NODES = 10000
NP = 10112
F1 = 16
NC, NS = 2, 16
NW = NC * NS
CB = 128
STRIPE = NP // NS


def _sc_compiler_params():
    cp = pltpu.CompilerParams()
    fields = pltpu.CompilerParams.__dataclass_fields__
    if "needs_layout_passes" in fields:
        cp = dataclasses.replace(cp, needs_layout_passes=False)
    if "use_tc_tiling_on_sc" in fields:
        cp = dataclasses.replace(cp, use_tc_tiling_on_sc=False)
    return cp


def _split_edges(ei):
    e = ei.shape[1]
    assert e % CB == 0
    rows = e // CB
    rt = -(-rows // (NW * 8)) * 8
    padr = rt * NW - rows
    pad1 = (N_NODES + (np.arange(max(padr, 1) * CB) % (NP - N_NODES)))
    pad3 = jnp.asarray(
        np.broadcast_to(pad1.reshape(1, -1, CB), (2, max(padr, 1), CB)),
        dtype=jnp.int32)
    return ei.reshape(2, rows, CB), pad3, rows, rt


def _stage_indices(ei_hbm, pad_hbm, wid, esrc, edst, rows, rt):
    full = rows // rt
    rem = rows % rt

    if full > 0:
        @pl.when(wid < full)
        def _():
            pltpu.sync_copy(ei_hbm.at[0, pl.ds(wid * rt, rt)], esrc)
            pltpu.sync_copy(ei_hbm.at[1, pl.ds(wid * rt, rt)], edst)

    if rem > 0:
        @pl.when(wid == full)
        def _():
            pltpu.sync_copy(ei_hbm.at[0, pl.ds(full * rt, rem)],
                            esrc.at[pl.ds(0, rem)])
            pltpu.sync_copy(ei_hbm.at[1, pl.ds(full * rt, rem)],
                            edst.at[pl.ds(0, rem)])
            pltpu.sync_copy(pad_hbm.at[0, pl.ds(0, rt - rem)],
                            esrc.at[pl.ds(rem, rt - rem)])
            pltpu.sync_copy(pad_hbm.at[1, pl.ds(0, rt - rem)],
                            edst.at[pl.ds(rem, rt - rem)])

    first_all_pad = full + (1 if rem else 0)
    if first_all_pad < NW:
        @pl.when(wid >= first_all_pad)
        def _():
            base = wid * rt - rows
            pltpu.sync_copy(pad_hbm.at[0, pl.ds(base, rt)], esrc)
            pltpu.sync_copy(pad_hbm.at[1, pl.ds(base, rt)], edst)


def _sc_degree(ei3, pad3, rows, rt):
    mesh = plsc.VectorSubcoreMesh(core_axis_name="c", subcore_axis_name="s")

    @functools.partial(
        pl.kernel,
        out_type=jax.ShapeDtypeStruct((NW, NP), jnp.float32),
        mesh=mesh,
        scratch_types=[
            pltpu.VMEM((NP,), jnp.float32),
            pltpu.VMEM((rt, CB), jnp.int32),
            pltpu.VMEM((rt, CB), jnp.int32),
        ],
        compiler_params=_sc_compiler_params(),
    )
    def deg_kernel(ei_hbm, pad_hbm, out_hbm, hist, esrc, edst):
        cid = lax.axis_index("c")
        sid = lax.axis_index("s")
        wid = cid * NS + sid

        @pl.loop(0, NP // 16)
        def _(i):
            hist[pl.ds(i * 16, 16)] = jnp.zeros((16,), jnp.float32)

        _stage_indices(ei_hbm, pad_hbm, wid, esrc, edst, rows, rt)

        ones = jnp.ones((16,), jnp.float32)

        @pl.loop(0, rt)
        def _(j):
            for buf in (esrc, edst):
                for k in range(CB // 16):
                    idx = buf[j, pl.ds(k * 16, 16)]
                    plsc.addupdate_scatter(hist, [idx], ones)

        pltpu.sync_copy(hist, out_hbm.at[wid])

    return deg_kernel(ei3, pad3)


def _sc_aggregate(y, ei3, pad3, rows, rt):
    mesh = plsc.VectorSubcoreMesh(core_axis_name="c", subcore_axis_name="s")

    @functools.partial(
        pl.kernel,
        out_type=jax.ShapeDtypeStruct((NC, NP, F1), jnp.float32),
        mesh=mesh,
        scratch_types=[
            pltpu.VMEM_SHARED((NP, F1), jnp.float32),
            pltpu.VMEM((rt, CB), jnp.int32),
            pltpu.VMEM((rt, CB), jnp.int32),
            *([pltpu.VMEM((CB, F1), jnp.float32)] * 16),
            pltpu.VMEM((STRIPE, F1), jnp.float32),
            *([pltpu.SemaphoreType.DMA] * 16),
        ],
        compiler_params=_sc_compiler_params(),
    )
    def agg_kernel(y_hbm, ei_hbm, pad_hbm, out_hbm, acc, esrc, edst,
                   *rest):
        cid = lax.axis_index("c")
        sid = lax.axis_index("s")
        wid = cid * NS + sid

        bufs = rest[:16]
        zbuf = rest[16]
        sems = rest[17:]
        MB, LEAD = 8, 4
        slots = tuple(
            (bufs[2 * b], bufs[2 * b + 1], sems[2 * b], sems[2 * b + 1])
            for b in range(MB))

        @pl.loop(0, STRIPE)
        def _(i):
            zbuf[i, :] = jnp.zeros((16,), jnp.float32)

        pltpu.sync_copy(zbuf, acc.at[pl.ds(sid * STRIPE, STRIPE)])
        plsc.subcore_barrier()

        _stage_indices(ei_hbm, pad_hbm, wid, esrc, edst, rows, rt)

        for b in range(LEAD):
            ra, rb, gs, _ = slots[b]
            pltpu.async_copy(y_hbm.at[esrc.at[b]], ra, gs)
            pltpu.async_copy(y_hbm.at[edst.at[b]], rb, gs)

        @pl.loop(0, rt, step=MB)
        def _(j):
            for b in range(MB):
                jc = j + b
                ra, rb, gs, ss = slots[b]
                pa, pb, pgs, pss = slots[(b + LEAD) % MB]
                jp = jc + LEAD

                @pl.when(jp < rt)
                def _():
                    @pl.when(jp >= MB)
                    def _():
                        pltpu.make_async_copy(
                            pa, acc.at[edst.at[jp - MB]], pss).wait()
                        pltpu.make_async_copy(
                            pb, acc.at[esrc.at[jp - MB]], pss).wait()
                    pltpu.async_copy(y_hbm.at[esrc.at[jp]], pa, pgs)
                    pltpu.async_copy(y_hbm.at[edst.at[jp]], pb, pgs)

                pltpu.make_async_copy(y_hbm.at[esrc.at[jc]], ra, gs).wait()
                pltpu.make_async_copy(y_hbm.at[edst.at[jc]], rb, gs).wait()
                pltpu.async_copy(ra, acc.at[edst.at[jc]], ss, add=True)
                pltpu.async_copy(rb, acc.at[esrc.at[jc]], ss, add=True)

        for b in range(MB):
            ra, rb, _, ss = slots[b]
            jc = rt - MB + b
            pltpu.make_async_copy(ra, acc.at[edst.at[jc]], ss).wait()
            pltpu.make_async_copy(rb, acc.at[esrc.at[jc]], ss).wait()

        plsc.subcore_barrier()
        pltpu.sync_copy(acc.at[pl.ds(sid * STRIPE, STRIPE)],
                        out_hbm.at[cid, pl.ds(sid * STRIPE, STRIPE)])

    return agg_kernel(y, ei3, pad3)


def _tc_project(x, w):
    n, kdim = x.shape
    f = w.shape[1]
    nb = 5
    bs = n // nb

    def body(x_ref, w_ref, o_ref):
        o_ref[...] = jnp.dot(x_ref[...], w_ref[...],
                             preferred_element_type=jnp.float32,
                             precision=lax.Precision.HIGHEST)

    return pl.pallas_call(
        body,
        grid=(nb,),
        in_specs=[pl.BlockSpec((bs, kdim), lambda i: (i, 0)),
                  pl.BlockSpec((kdim, f), lambda i: (0, 0))],
        out_specs=pl.BlockSpec((bs, f), lambda i: (i, 0)),
        out_shape=jax.ShapeDtypeStruct((n, f), jnp.float32),
    )(x, w)


def _tc_norm_scale(partials, u):
    def body(p_ref, u_ref, y_ref, n_ref):
        ones = jnp.ones((NW, 1), jnp.float32)
        deg = lax.dot_general(p_ref[...], ones, (((0,), (0,)), ((), ())),
                              preferred_element_type=jnp.float32)
        nc = lax.rsqrt(jnp.maximum(deg, 1.0))
        n_ref[...] = nc
        y_ref[pl.ds(0, N_NODES), :] = u_ref[...] * nc[:N_NODES, :]
        y_ref[pl.ds(N_NODES, NP - N_NODES), :] = jnp.zeros(
            (NP - N_NODES, F1), jnp.float32)

    return pl.pallas_call(
        body,
        out_shape=(jax.ShapeDtypeStruct((NP, F1), jnp.float32),
                   jax.ShapeDtypeStruct((NP, 1), jnp.float32)),
    )(partials, u)


def _tc_mid(parts, norm_col, b1):
    def body(p_ref, n_ref, b_ref, o_ref):
        agg = p_ref[0] + p_ref[1]
        nrm = n_ref[...]
        t = agg * nrm + b_ref[...]
        h = jnp.where(t > 0, t, jnp.exp(t) - 1.0)
        o_ref[...] = h * nrm

    return pl.pallas_call(
        body, out_shape=jax.ShapeDtypeStruct((NP, F1), jnp.float32),
    )(parts, norm_col, b1)


def _tc_final(parts, norm_col, w2, b2):
    f2 = w2.shape[1]

    def body(q_ref, n_ref, w_ref, b_ref, o_ref):
        agg = q_ref[0, pl.ds(0, N_NODES), :] + q_ref[1, pl.ds(0, N_NODES), :]
        agg = agg * n_ref[pl.ds(0, N_NODES), :]
        o_ref[...] = jnp.dot(agg, w_ref[...],
                             preferred_element_type=jnp.float32,
                             precision=lax.Precision.HIGHEST) + b_ref[...]

    return pl.pallas_call(
        body, out_shape=jax.ShapeDtypeStruct((N_NODES, f2), jnp.float32),
    )(parts, norm_col, w2, b2)


def kernel(x, edge_index, W1, b1, W2, b2):
    ei3, pad3, rows, rt = _split_edges(edge_index.astype(jnp.int32))
    partials = _sc_degree(ei3, pad3, rows, rt)
    u1 = _tc_project(x, W1)
    y, norm_col = _tc_norm_scale(partials, u1)
    p1 = _sc_aggregate(y, ei3, pad3, rows, rt)
    z = _tc_mid(p1, norm_col, jnp.reshape(b1, (1, F1)))
    p2 = _sc_aggregate(z, ei3, pad3, rows, rt)
    return _tc_final(p2, norm_col, W2, jnp.reshape(b2, (1, -1)))


# speedup vs baseline: 1.0142x; 1.0142x over previous
import dataclasses
import functools

import jax
import jax.numpy as jnp
import numpy as np
from jax import lax
from jax.experimental import pallas as pl
from jax.experimental.pallas import tpu as pltpu
from jax.experimental.pallas import tpu_sc as plsc

N_NODES = 10000
NP = 10112
F1 = 16
NC, NS = 2, 16
NW = NC * NS
CB = 128
STRIPE = NP // NS


def _sc_compiler_params():
    cp = pltpu.CompilerParams()
    fields = pltpu.CompilerParams.__dataclass_fields__
    if "needs_layout_passes" in fields:
        cp = dataclasses.replace(cp, needs_layout_passes=False)
    if "use_tc_tiling_on_sc" in fields:
        cp = dataclasses.replace(cp, use_tc_tiling_on_sc=False)
    return cp


def _split_edges(ei):
    e = ei.shape[1]
    assert e % CB == 0
    rows = e // CB
    rt = -(-rows // (NW * 8)) * 8
    padr = rt * NW - rows
    pad1 = (N_NODES + (np.arange(max(padr, 1) * CB) % (NP - N_NODES)))
    pad3 = jnp.asarray(
        np.broadcast_to(pad1.reshape(1, -1, CB), (2, max(padr, 1), CB)),
        dtype=jnp.int32)
    return ei.reshape(2, rows, CB), pad3, rows, rt


def _stage_indices(ei_hbm, pad_hbm, wid, esrc, edst, rows, rt):
    full = rows // rt
    rem = rows % rt

    if full > 0:
        @pl.when(wid < full)
        def _():
            pltpu.sync_copy(ei_hbm.at[0, pl.ds(wid * rt, rt)], esrc)
            pltpu.sync_copy(ei_hbm.at[1, pl.ds(wid * rt, rt)], edst)

    if rem > 0:
        @pl.when(wid == full)
        def _():
            pltpu.sync_copy(ei_hbm.at[0, pl.ds(full * rt, rem)],
                            esrc.at[pl.ds(0, rem)])
            pltpu.sync_copy(ei_hbm.at[1, pl.ds(full * rt, rem)],
                            edst.at[pl.ds(0, rem)])
            pltpu.sync_copy(pad_hbm.at[0, pl.ds(0, rt - rem)],
                            esrc.at[pl.ds(rem, rt - rem)])
            pltpu.sync_copy(pad_hbm.at[1, pl.ds(0, rt - rem)],
                            edst.at[pl.ds(rem, rt - rem)])

    first_all_pad = full + (1 if rem else 0)
    if first_all_pad < NW:
        @pl.when(wid >= first_all_pad)
        def _():
            base = wid * rt - rows
            pltpu.sync_copy(pad_hbm.at[0, pl.ds(base, rt)], esrc)
            pltpu.sync_copy(pad_hbm.at[1, pl.ds(base, rt)], edst)


def _sc_degree(ei3, pad3, rows, rt):
    mesh = plsc.VectorSubcoreMesh(core_axis_name="c", subcore_axis_name="s")

    @functools.partial(
        pl.kernel,
        out_type=jax.ShapeDtypeStruct((NW, NP), jnp.float32),
        mesh=mesh,
        scratch_types=[
            pltpu.VMEM((NP,), jnp.float32),
            pltpu.VMEM((rt, CB), jnp.int32),
            pltpu.VMEM((rt, CB), jnp.int32),
        ],
        compiler_params=_sc_compiler_params(),
    )
    def deg_kernel(ei_hbm, pad_hbm, out_hbm, hist, esrc, edst):
        cid = lax.axis_index("c")
        sid = lax.axis_index("s")
        wid = cid * NS + sid

        @pl.loop(0, NP // 16)
        def _(i):
            hist[pl.ds(i * 16, 16)] = jnp.zeros((16,), jnp.float32)

        _stage_indices(ei_hbm, pad_hbm, wid, esrc, edst, rows, rt)

        ones = jnp.ones((16,), jnp.float32)

        @pl.loop(0, rt)
        def _(j):
            for buf in (esrc, edst):
                for k in range(CB // 16):
                    idx = buf[j, pl.ds(k * 16, 16)]
                    plsc.addupdate_scatter(hist, [idx], ones)

        pltpu.sync_copy(hist, out_hbm.at[wid])

    return deg_kernel(ei3, pad3)


def _sc_aggregate(y, ei3, pad3, rows, rt):
    mesh = plsc.VectorSubcoreMesh(core_axis_name="c", subcore_axis_name="s")

    @functools.partial(
        pl.kernel,
        out_type=jax.ShapeDtypeStruct((NC, NP, F1), jnp.float32),
        mesh=mesh,
        scratch_types=[
            pltpu.VMEM_SHARED((NP, F1), jnp.float32),
            pltpu.VMEM((rt, CB), jnp.int32),
            pltpu.VMEM((rt, CB), jnp.int32),
            *([pltpu.VMEM((CB, F1), jnp.float32)] * 16),
            pltpu.VMEM((STRIPE, F1), jnp.float32),
            *([pltpu.SemaphoreType.DMA] * 16),
        ],
        compiler_params=_sc_compiler_params(),
    )
    def agg_kernel(y_hbm, ei_hbm, pad_hbm, out_hbm, acc, esrc, edst,
                   *rest):
        cid = lax.axis_index("c")
        sid = lax.axis_index("s")
        wid = cid * NS + sid

        bufs = rest[:16]
        zbuf = rest[16]
        sems = rest[17:]
        nbuf = 4
        slots = tuple((bufs[2 * b], bufs[2 * b + 1], sems[b])
                      for b in range(nbuf))

        @pl.loop(0, STRIPE)
        def _(i):
            zbuf[i, :] = jnp.zeros((16,), jnp.float32)

        pltpu.sync_copy(zbuf, acc.at[pl.ds(sid * STRIPE, STRIPE)])
        plsc.subcore_barrier()

        _stage_indices(ei_hbm, pad_hbm, wid, esrc, edst, rows, rt)

        for b in range(nbuf):
            ra, rb, sem = slots[b]
            pltpu.async_copy(y_hbm.at[esrc.at[b]], ra, sem)
            pltpu.async_copy(y_hbm.at[edst.at[b]], rb, sem)

        @pl.loop(0, rt, step=nbuf)
        def _(j):
            for b in range(nbuf):
                ra, rb, sem = slots[b]
                jc = j + b
                pltpu.make_async_copy(y_hbm.at[esrc.at[jc]], ra, sem).wait()
                pltpu.make_async_copy(y_hbm.at[edst.at[jc]], rb, sem).wait()
                pltpu.sync_copy(ra, acc.at[edst.at[jc]], add=True)
                pltpu.sync_copy(rb, acc.at[esrc.at[jc]], add=True)

                @pl.when(jc + nbuf < rt)
                def _():
                    pltpu.async_copy(y_hbm.at[esrc.at[jc + nbuf]], ra, sem)
                    pltpu.async_copy(y_hbm.at[edst.at[jc + nbuf]], rb, sem)

        plsc.subcore_barrier()
        pltpu.sync_copy(acc.at[pl.ds(sid * STRIPE, STRIPE)],
                        out_hbm.at[cid, pl.ds(sid * STRIPE, STRIPE)])

    return agg_kernel(y, ei3, pad3)


def _tc_project(x, w):
    n, kdim = x.shape
    f = w.shape[1]
    nb = 5
    bs = n // nb

    def body(x_ref, w_ref, o_ref):
        o_ref[...] = jnp.dot(x_ref[...], w_ref[...],
                             preferred_element_type=jnp.float32,
                             precision=lax.Precision.HIGHEST)

    return pl.pallas_call(
        body,
        grid=(nb,),
        in_specs=[pl.BlockSpec((bs, kdim), lambda i: (i, 0)),
                  pl.BlockSpec((kdim, f), lambda i: (0, 0))],
        out_specs=pl.BlockSpec((bs, f), lambda i: (i, 0)),
        out_shape=jax.ShapeDtypeStruct((n, f), jnp.float32),
    )(x, w)


def _tc_norm_scale(partials, u):
    def body(p_ref, u_ref, y_ref, n_ref):
        ones = jnp.ones((NW, 1), jnp.float32)
        deg = lax.dot_general(p_ref[...], ones, (((0,), (0,)), ((), ())),
                              preferred_element_type=jnp.float32)
        nc = lax.rsqrt(jnp.maximum(deg, 1.0))
        n_ref[...] = nc
        y_ref[pl.ds(0, N_NODES), :] = u_ref[...] * nc[:N_NODES, :]
        y_ref[pl.ds(N_NODES, NP - N_NODES), :] = jnp.zeros(
            (NP - N_NODES, F1), jnp.float32)

    return pl.pallas_call(
        body,
        out_shape=(jax.ShapeDtypeStruct((NP, F1), jnp.float32),
                   jax.ShapeDtypeStruct((NP, 1), jnp.float32)),
    )(partials, u)


def _tc_mid(parts, norm_col, b1):
    def body(p_ref, n_ref, b_ref, o_ref):
        agg = p_ref[0] + p_ref[1]
        nrm = n_ref[...]
        t = agg * nrm + b_ref[...]
        h = jnp.where(t > 0, t, jnp.exp(t) - 1.0)
        o_ref[...] = h * nrm

    return pl.pallas_call(
        body, out_shape=jax.ShapeDtypeStruct((NP, F1), jnp.float32),
    )(parts, norm_col, b1)


def _tc_final(parts, norm_col, w2, b2):
    f2 = w2.shape[1]

    def body(q_ref, n_ref, w_ref, b_ref, o_ref):
        agg = q_ref[0, pl.ds(0, N_NODES), :] + q_ref[1, pl.ds(0, N_NODES), :]
        agg = agg * n_ref[pl.ds(0, N_NODES), :]
        o_ref[...] = jnp.dot(agg, w_ref[...],
                             preferred_element_type=jnp.float32,
                             precision=lax.Precision.HIGHEST) + b_ref[...]

    return pl.pallas_call(
        body, out_shape=jax.ShapeDtypeStruct((N_NODES, f2), jnp.float32),
    )(parts, norm_col, w2, b2)


def kernel(x, edge_index, W1, b1, W2, b2):
    ei3, pad3, rows, rt = _split_edges(edge_index.astype(jnp.int32))
    partials = _sc_degree(ei3, pad3, rows, rt)
    u1 = _tc_project(x, W1)
    y, norm_col = _tc_norm_scale(partials, u1)
    p1 = _sc_aggregate(y, ei3, pad3, rows, rt)
    z = _tc_mid(p1, norm_col, jnp.reshape(b1, (1, F1)))
    p2 = _sc_aggregate(z, ei3, pad3, rows, rt)
    return _tc_final(p2, norm_col, W2, jnp.reshape(b2, (1, -1)))


# speedup vs baseline: 1.0264x; 1.0120x over previous
import dataclasses
import functools

import jax
import jax.numpy as jnp
import numpy as np
from jax import lax
from jax.experimental import pallas as pl
from jax.experimental.pallas import tpu as pltpu
from jax.experimental.pallas import tpu_sc as plsc

N_NODES = 10000
NP = 10112
F1 = 16
NC, NS = 2, 16
NW = NC * NS
CB = 128
STRIPE = NP // NS


def _sc_compiler_params():
    cp = pltpu.CompilerParams()
    fields = pltpu.CompilerParams.__dataclass_fields__
    if "needs_layout_passes" in fields:
        cp = dataclasses.replace(cp, needs_layout_passes=False)
    if "use_tc_tiling_on_sc" in fields:
        cp = dataclasses.replace(cp, use_tc_tiling_on_sc=False)
    return cp


def _split_edges(ei):
    e = ei.shape[1]
    assert e % CB == 0
    rows = e // CB
    rt = -(-rows // (NW * 8)) * 8
    padr = rt * NW - rows
    pad1 = (N_NODES + (np.arange(max(padr, 1) * CB) % (NP - N_NODES)))
    pad3 = jnp.asarray(
        np.broadcast_to(pad1.reshape(1, -1, CB), (2, max(padr, 1), CB)),
        dtype=jnp.int32)
    return ei.reshape(2, rows, CB), pad3, rows, rt


def _stage_indices(ei_hbm, pad_hbm, wid, esrc, edst, rows, rt):
    full = rows // rt
    rem = rows % rt

    if full > 0:
        @pl.when(wid < full)
        def _():
            pltpu.sync_copy(ei_hbm.at[0, pl.ds(wid * rt, rt)], esrc)
            pltpu.sync_copy(ei_hbm.at[1, pl.ds(wid * rt, rt)], edst)

    if rem > 0:
        @pl.when(wid == full)
        def _():
            pltpu.sync_copy(ei_hbm.at[0, pl.ds(full * rt, rem)],
                            esrc.at[pl.ds(0, rem)])
            pltpu.sync_copy(ei_hbm.at[1, pl.ds(full * rt, rem)],
                            edst.at[pl.ds(0, rem)])
            pltpu.sync_copy(pad_hbm.at[0, pl.ds(0, rt - rem)],
                            esrc.at[pl.ds(rem, rt - rem)])
            pltpu.sync_copy(pad_hbm.at[1, pl.ds(0, rt - rem)],
                            edst.at[pl.ds(rem, rt - rem)])

    first_all_pad = full + (1 if rem else 0)
    if first_all_pad < NW:
        @pl.when(wid >= first_all_pad)
        def _():
            base = wid * rt - rows
            pltpu.sync_copy(pad_hbm.at[0, pl.ds(base, rt)], esrc)
            pltpu.sync_copy(pad_hbm.at[1, pl.ds(base, rt)], edst)


def _sc_degree(ei3, pad3, rows, rt):
    mesh = plsc.VectorSubcoreMesh(core_axis_name="c", subcore_axis_name="s")

    @functools.partial(
        pl.kernel,
        out_type=jax.ShapeDtypeStruct((NW, NP), jnp.float32),
        mesh=mesh,
        scratch_types=[
            pltpu.VMEM((NP,), jnp.float32),
            pltpu.VMEM((rt, CB), jnp.int32),
            pltpu.VMEM((rt, CB), jnp.int32),
        ],
        compiler_params=_sc_compiler_params(),
    )
    def deg_kernel(ei_hbm, pad_hbm, out_hbm, hist, esrc, edst):
        cid = lax.axis_index("c")
        sid = lax.axis_index("s")
        wid = cid * NS + sid

        @pl.loop(0, NP // 16)
        def _(i):
            hist[pl.ds(i * 16, 16)] = jnp.zeros((16,), jnp.float32)

        _stage_indices(ei_hbm, pad_hbm, wid, esrc, edst, rows, rt)

        ones = jnp.ones((16,), jnp.float32)

        @pl.loop(0, rt)
        def _(j):
            for buf in (esrc, edst):
                for k in range(CB // 16):
                    idx = buf[j, pl.ds(k * 16, 16)]
                    plsc.addupdate_scatter(hist, [idx], ones)

        pltpu.sync_copy(hist, out_hbm.at[wid])

    return deg_kernel(ei3, pad3)


def _sc_aggregate(y, ei3, pad3, rows, rt):
    mesh = plsc.VectorSubcoreMesh(core_axis_name="c", subcore_axis_name="s")

    @functools.partial(
        pl.kernel,
        out_type=jax.ShapeDtypeStruct((NC, NP, F1), jnp.float32),
        mesh=mesh,
        scratch_types=[
            pltpu.VMEM_SHARED((NP, F1), jnp.float32),
            pltpu.VMEM((rt, CB), jnp.int32),
            pltpu.VMEM((rt, CB), jnp.int32),
            *([pltpu.VMEM((CB, F1), jnp.float32)] * 8),
            pltpu.VMEM((STRIPE, F1), jnp.float32),
            *([pltpu.SemaphoreType.DMA] * 8),
        ],
        compiler_params=_sc_compiler_params(),
    )
    def agg_kernel(y_hbm, ei_hbm, pad_hbm, out_hbm, acc, esrc, edst,
                   *rest):
        cid = lax.axis_index("c")
        sid = lax.axis_index("s")
        wid = cid * NS + sid

        bufs = rest[:8]
        zbuf = rest[8]
        sems = rest[9:]
        nbuf = 4
        slots = tuple((bufs[2 * b], bufs[2 * b + 1],
                       sems[2 * b], sems[2 * b + 1])
                      for b in range(nbuf))

        @pl.loop(0, STRIPE)
        def _(i):
            zbuf[i, :] = jnp.zeros((16,), jnp.float32)

        pltpu.sync_copy(zbuf, acc.at[pl.ds(sid * STRIPE, STRIPE)])
        plsc.subcore_barrier()

        _stage_indices(ei_hbm, pad_hbm, wid, esrc, edst, rows, rt)

        for b in range(nbuf):
            ra, rb, sa, sb = slots[b]
            pltpu.async_copy(y_hbm.at[esrc.at[b]], ra, sa)
            pltpu.async_copy(y_hbm.at[edst.at[b]], rb, sb)

        @pl.loop(0, rt, step=nbuf)
        def _(j):
            for b in range(nbuf):
                ra, rb, sa, sb = slots[b]
                jc = j + b
                pltpu.make_async_copy(y_hbm.at[esrc.at[jc]], ra, sa).wait()
                pltpu.sync_copy(ra, acc.at[edst.at[jc]], add=True)
                pltpu.make_async_copy(y_hbm.at[edst.at[jc]], rb, sb).wait()
                pltpu.sync_copy(rb, acc.at[esrc.at[jc]], add=True)

                @pl.when(jc + nbuf < rt)
                def _():
                    pltpu.async_copy(y_hbm.at[esrc.at[jc + nbuf]], ra, sa)
                    pltpu.async_copy(y_hbm.at[edst.at[jc + nbuf]], rb, sb)

        plsc.subcore_barrier()
        pltpu.sync_copy(acc.at[pl.ds(sid * STRIPE, STRIPE)],
                        out_hbm.at[cid, pl.ds(sid * STRIPE, STRIPE)])

    return agg_kernel(y, ei3, pad3)


def _tc_project(x, w):
    n, kdim = x.shape
    f = w.shape[1]
    nb = 5
    bs = n // nb

    def body(x_ref, w_ref, o_ref):
        o_ref[...] = jnp.dot(x_ref[...], w_ref[...],
                             preferred_element_type=jnp.float32,
                             precision=lax.Precision.HIGHEST)

    return pl.pallas_call(
        body,
        grid=(nb,),
        in_specs=[pl.BlockSpec((bs, kdim), lambda i: (i, 0)),
                  pl.BlockSpec((kdim, f), lambda i: (0, 0))],
        out_specs=pl.BlockSpec((bs, f), lambda i: (i, 0)),
        out_shape=jax.ShapeDtypeStruct((n, f), jnp.float32),
    )(x, w)


def _tc_norm_scale(partials, u):
    def body(p_ref, u_ref, y_ref, n_ref):
        ones = jnp.ones((NW, 1), jnp.float32)
        deg = lax.dot_general(p_ref[...], ones, (((0,), (0,)), ((), ())),
                              preferred_element_type=jnp.float32)
        nc = lax.rsqrt(jnp.maximum(deg, 1.0))
        n_ref[...] = nc
        y_ref[pl.ds(0, N_NODES), :] = u_ref[...] * nc[:N_NODES, :]
        y_ref[pl.ds(N_NODES, NP - N_NODES), :] = jnp.zeros(
            (NP - N_NODES, F1), jnp.float32)

    return pl.pallas_call(
        body,
        out_shape=(jax.ShapeDtypeStruct((NP, F1), jnp.float32),
                   jax.ShapeDtypeStruct((NP, 1), jnp.float32)),
    )(partials, u)


def _tc_mid(parts, norm_col, b1):
    def body(p_ref, n_ref, b_ref, o_ref):
        agg = p_ref[0] + p_ref[1]
        nrm = n_ref[...]
        t = agg * nrm + b_ref[...]
        h = jnp.where(t > 0, t, jnp.exp(t) - 1.0)
        o_ref[...] = h * nrm

    return pl.pallas_call(
        body, out_shape=jax.ShapeDtypeStruct((NP, F1), jnp.float32),
    )(parts, norm_col, b1)


def _tc_final(parts, norm_col, w2, b2):
    f2 = w2.shape[1]

    def body(q_ref, n_ref, w_ref, b_ref, o_ref):
        agg = q_ref[0, pl.ds(0, N_NODES), :] + q_ref[1, pl.ds(0, N_NODES), :]
        agg = agg * n_ref[pl.ds(0, N_NODES), :]
        o_ref[...] = jnp.dot(agg, w_ref[...],
                             preferred_element_type=jnp.float32,
                             precision=lax.Precision.HIGHEST) + b_ref[...]

    return pl.pallas_call(
        body, out_shape=jax.ShapeDtypeStruct((N_NODES, f2), jnp.float32),
    )(parts, norm_col, w2, b2)


def kernel(x, edge_index, W1, b1, W2, b2):
    ei3, pad3, rows, rt = _split_edges(edge_index.astype(jnp.int32))
    partials = _sc_degree(ei3, pad3, rows, rt)
    u1 = _tc_project(x, W1)
    y, norm_col = _tc_norm_scale(partials, u1)
    p1 = _sc_aggregate(y, ei3, pad3, rows, rt)
    z = _tc_mid(p1, norm_col, jnp.reshape(b1, (1, F1)))
    p2 = _sc_aggregate(z, ei3, pad3, rows, rt)
    return _tc_final(p2, norm_col, W2, jnp.reshape(b2, (1, -1)))


# speedup vs baseline: 1.0539x; 1.0268x over previous
import dataclasses
import functools

import jax
import jax.numpy as jnp
import numpy as np
from jax import lax
from jax.experimental import pallas as pl
from jax.experimental.pallas import tpu as pltpu
from jax.experimental.pallas import tpu_sc as plsc

N_NODES = 10000
NP = 10112
F1 = 16
NC, NS = 2, 16
NW = NC * NS
CB = 128
STRIPE = NP // NS


def _sc_compiler_params():
    cp = pltpu.CompilerParams()
    fields = pltpu.CompilerParams.__dataclass_fields__
    if "needs_layout_passes" in fields:
        cp = dataclasses.replace(cp, needs_layout_passes=False)
    if "use_tc_tiling_on_sc" in fields:
        cp = dataclasses.replace(cp, use_tc_tiling_on_sc=False)
    return cp


def _split_edges(ei):
    e = ei.shape[1]
    assert e % CB == 0
    rows = e // CB
    rt = -(-rows // (NW * 8)) * 8
    padr = rt * NW - rows
    pad1 = (N_NODES + (np.arange(max(padr, 1) * CB) % (NP - N_NODES)))
    pad3 = jnp.asarray(
        np.broadcast_to(pad1.reshape(1, -1, CB), (2, max(padr, 1), CB)),
        dtype=jnp.int32)
    return ei.reshape(2, rows, CB), pad3, rows, rt


def _stage_indices(ei_hbm, pad_hbm, wid, esrc, edst, rows, rt,
                   sem, do_wait):
    full = rows // rt
    rem = rows % rt

    def emit(src, dst):
        if do_wait:
            pltpu.make_async_copy(src, dst, sem).wait()
        else:
            pltpu.async_copy(src, dst, sem)

    if full > 0:
        @pl.when(wid < full)
        def _():
            emit(ei_hbm.at[0, pl.ds(wid * rt, rt)], esrc)
            emit(ei_hbm.at[1, pl.ds(wid * rt, rt)], edst)

    if rem > 0:
        @pl.when(wid == full)
        def _():
            emit(ei_hbm.at[0, pl.ds(full * rt, rem)],
                 esrc.at[pl.ds(0, rem)])
            emit(ei_hbm.at[1, pl.ds(full * rt, rem)],
                 edst.at[pl.ds(0, rem)])
            emit(pad_hbm.at[0, pl.ds(0, rt - rem)],
                 esrc.at[pl.ds(rem, rt - rem)])
            emit(pad_hbm.at[1, pl.ds(0, rt - rem)],
                 edst.at[pl.ds(rem, rt - rem)])

    first_all_pad = full + (1 if rem else 0)
    if first_all_pad < NW:
        @pl.when(wid >= first_all_pad)
        def _():
            base = wid * rt - rows
            emit(pad_hbm.at[0, pl.ds(base, rt)], esrc)
            emit(pad_hbm.at[1, pl.ds(base, rt)], edst)


def _sc_degree(ei3, pad3, rows, rt):
    mesh = plsc.VectorSubcoreMesh(core_axis_name="c", subcore_axis_name="s")

    @functools.partial(
        pl.kernel,
        out_type=jax.ShapeDtypeStruct((NW, NP), jnp.float32),
        mesh=mesh,
        scratch_types=[
            pltpu.VMEM((NP,), jnp.float32),
            pltpu.VMEM((rt, CB), jnp.int32),
            pltpu.VMEM((rt, CB), jnp.int32),
            pltpu.SemaphoreType.DMA,
        ],
        compiler_params=_sc_compiler_params(),
    )
    def deg_kernel(ei_hbm, pad_hbm, out_hbm, hist, esrc, edst, ssem):
        cid = lax.axis_index("c")
        sid = lax.axis_index("s")
        wid = cid * NS + sid

        _stage_indices(ei_hbm, pad_hbm, wid, esrc, edst, rows, rt,
                       ssem, False)

        @pl.loop(0, NP // 16)
        def _(i):
            hist[pl.ds(i * 16, 16)] = jnp.zeros((16,), jnp.float32)

        _stage_indices(ei_hbm, pad_hbm, wid, esrc, edst, rows, rt,
                       ssem, True)

        ones = jnp.ones((16,), jnp.float32)

        @pl.loop(0, rt)
        def _(j):
            for buf in (esrc, edst):
                for k in range(CB // 16):
                    idx = buf[j, pl.ds(k * 16, 16)]
                    plsc.addupdate_scatter(hist, [idx], ones)

        pltpu.sync_copy(hist, out_hbm.at[wid])

    return deg_kernel(ei3, pad3)


def _sc_aggregate(y, ei3, pad3, rows, rt):
    mesh = plsc.VectorSubcoreMesh(core_axis_name="c", subcore_axis_name="s")

    @functools.partial(
        pl.kernel,
        out_type=jax.ShapeDtypeStruct((NC, NP, F1), jnp.float32),
        mesh=mesh,
        scratch_types=[
            pltpu.VMEM_SHARED((NP, F1), jnp.float32),
            pltpu.VMEM((rt, CB), jnp.int32),
            pltpu.VMEM((rt, CB), jnp.int32),
            *([pltpu.VMEM((CB, F1), jnp.float32)] * 8),
            pltpu.VMEM((STRIPE, F1), jnp.float32),
            *([pltpu.SemaphoreType.DMA] * 9),
        ],
        compiler_params=_sc_compiler_params(),
    )
    def agg_kernel(y_hbm, ei_hbm, pad_hbm, out_hbm, acc, esrc, edst,
                   *rest):
        cid = lax.axis_index("c")
        sid = lax.axis_index("s")
        wid = cid * NS + sid

        bufs = rest[:8]
        zbuf = rest[8]
        sems = rest[9:]
        nbuf = 4
        slots = tuple((bufs[2 * b], bufs[2 * b + 1],
                       sems[2 * b], sems[2 * b + 1])
                      for b in range(nbuf))

        ssem = sems[8]
        _stage_indices(ei_hbm, pad_hbm, wid, esrc, edst, rows, rt,
                       ssem, False)

        @pl.loop(0, STRIPE)
        def _(i):
            zbuf[i, :] = jnp.zeros((16,), jnp.float32)

        pltpu.sync_copy(zbuf, acc.at[pl.ds(sid * STRIPE, STRIPE)])
        plsc.subcore_barrier()

        _stage_indices(ei_hbm, pad_hbm, wid, esrc, edst, rows, rt,
                       ssem, True)

        for b in range(nbuf):
            ra, rb, sa, sb = slots[b]
            pltpu.async_copy(y_hbm.at[esrc.at[b]], ra, sa)
            pltpu.async_copy(y_hbm.at[edst.at[b]], rb, sb)

        @pl.loop(0, rt, step=nbuf)
        def _(j):
            for b in range(nbuf):
                ra, rb, sa, sb = slots[b]
                jc = j + b
                pltpu.make_async_copy(y_hbm.at[esrc.at[jc]], ra, sa).wait()
                pltpu.sync_copy(ra, acc.at[edst.at[jc]], add=True)
                pltpu.make_async_copy(y_hbm.at[edst.at[jc]], rb, sb).wait()
                pltpu.sync_copy(rb, acc.at[esrc.at[jc]], add=True)

                @pl.when(jc + nbuf < rt)
                def _():
                    pltpu.async_copy(y_hbm.at[esrc.at[jc + nbuf]], ra, sa)
                    pltpu.async_copy(y_hbm.at[edst.at[jc + nbuf]], rb, sb)

        plsc.subcore_barrier()
        pltpu.sync_copy(acc.at[pl.ds(sid * STRIPE, STRIPE)],
                        out_hbm.at[cid, pl.ds(sid * STRIPE, STRIPE)])

    return agg_kernel(y, ei3, pad3)


def _tc_project(x, w):
    n, kdim = x.shape
    f = w.shape[1]
    nb = 5
    bs = n // nb

    def body(x_ref, w_ref, o_ref):
        o_ref[...] = jnp.dot(x_ref[...], w_ref[...],
                             preferred_element_type=jnp.float32,
                             precision=lax.Precision.HIGHEST)

    return pl.pallas_call(
        body,
        grid=(nb,),
        in_specs=[pl.BlockSpec((bs, kdim), lambda i: (i, 0)),
                  pl.BlockSpec((kdim, f), lambda i: (0, 0))],
        out_specs=pl.BlockSpec((bs, f), lambda i: (i, 0)),
        out_shape=jax.ShapeDtypeStruct((n, f), jnp.float32),
    )(x, w)


def _tc_norm_scale(partials, u):
    def body(p_ref, u_ref, y_ref, n_ref):
        ones = jnp.ones((NW, 1), jnp.float32)
        deg = lax.dot_general(p_ref[...], ones, (((0,), (0,)), ((), ())),
                              preferred_element_type=jnp.float32)
        nc = lax.rsqrt(jnp.maximum(deg, 1.0))
        n_ref[...] = nc
        y_ref[pl.ds(0, N_NODES), :] = u_ref[...] * nc[:N_NODES, :]
        y_ref[pl.ds(N_NODES, NP - N_NODES), :] = jnp.zeros(
            (NP - N_NODES, F1), jnp.float32)

    return pl.pallas_call(
        body,
        out_shape=(jax.ShapeDtypeStruct((NP, F1), jnp.float32),
                   jax.ShapeDtypeStruct((NP, 1), jnp.float32)),
    )(partials, u)


def _tc_mid(parts, norm_col, b1):
    def body(p_ref, n_ref, b_ref, o_ref):
        agg = p_ref[0] + p_ref[1]
        nrm = n_ref[...]
        t = agg * nrm + b_ref[...]
        h = jnp.where(t > 0, t, jnp.exp(t) - 1.0)
        o_ref[...] = h * nrm

    return pl.pallas_call(
        body, out_shape=jax.ShapeDtypeStruct((NP, F1), jnp.float32),
    )(parts, norm_col, b1)


def _tc_final(parts, norm_col, w2, b2):
    f2 = w2.shape[1]

    def body(q_ref, n_ref, w_ref, b_ref, o_ref):
        agg = q_ref[0, pl.ds(0, N_NODES), :] + q_ref[1, pl.ds(0, N_NODES), :]
        agg = agg * n_ref[pl.ds(0, N_NODES), :]
        o_ref[...] = jnp.dot(agg, w_ref[...],
                             preferred_element_type=jnp.float32,
                             precision=lax.Precision.HIGHEST) + b_ref[...]

    return pl.pallas_call(
        body, out_shape=jax.ShapeDtypeStruct((N_NODES, f2), jnp.float32),
    )(parts, norm_col, w2, b2)


def kernel(x, edge_index, W1, b1, W2, b2):
    ei3, pad3, rows, rt = _split_edges(edge_index.astype(jnp.int32))
    partials = _sc_degree(ei3, pad3, rows, rt)
    u1 = _tc_project(x, W1)
    y, norm_col = _tc_norm_scale(partials, u1)
    p1 = _sc_aggregate(y, ei3, pad3, rows, rt)
    z = _tc_mid(p1, norm_col, jnp.reshape(b1, (1, F1)))
    p2 = _sc_aggregate(z, ei3, pad3, rows, rt)
    return _tc_final(p2, norm_col, W2, jnp.reshape(b2, (1, -1)))
